# Initial kernel scaffold; baseline (speedup 1.0000x reference)
import jax
import jax.numpy as jnp
from jax.experimental import pallas as pl


def kernel(emb_in, sum_weights):
    raise NotImplementedError("write your pallas kernel here")


# speedup vs baseline: 3.9762x; 3.9762x over previous
import functools

import jax
import jax.numpy as jnp
from jax import lax
from jax.experimental import pallas as pl
from jax.experimental.pallas import tpu as pltpu
from jax.experimental.pallas import tpu_sc as plsc

N = 8192
D = 64
K = 16
BR = 256
W = 128


def _topk_body(q_ref, k_ref, idx_ref, csq_ref):
    i = pl.program_id(0)
    km = k_ref[...]

    @pl.when(i == 0)
    def _():
        csq_ref[...] = jnp.sum(km * km, axis=1)[None, :]

    q = q_ref[...]
    rsq = jnp.sum(q * q, axis=1, keepdims=True)
    dot = lax.dot_general(
        q.astype(jnp.bfloat16), km.astype(jnp.bfloat16),
        (((1,), (1,)), ((), ())), preferred_element_type=jnp.float32)
    s = (rsq + csq_ref[...]) - 2.0 * dot
    col = lax.broadcasted_iota(jnp.int32, (BR, N), 1)
    row = i * BR + lax.broadcasted_iota(jnp.int32, (BR, N), 0)
    s = s + jnp.where(col == row, jnp.float32(1e9), jnp.float32(0.0))

    lane16 = lax.broadcasted_iota(jnp.int32, (BR, K), 1)
    inf = jnp.float32(jnp.inf)

    def step(t, carry):
        m_prev, a_prev, acc = carry
        newer = (s > m_prev) | ((s == m_prev) & (col > a_prev))
        cand = jnp.where(newer, s, inf)
        m = jnp.min(cand, axis=1, keepdims=True)
        a = jnp.min(jnp.where(cand == m, col, jnp.int32(N)), axis=1,
                    keepdims=True)
        acc = jnp.where(lane16 == t, a, acc)
        return m, a, acc

    m0 = jnp.full((BR, 1), -inf, jnp.float32)
    a0 = jnp.full((BR, 1), -1, jnp.int32)
    acc0 = jnp.zeros((BR, K), jnp.int32)
    _, _, acc = lax.fori_loop(0, K - 1, step, (m0, a0, acc0))
    rowv = i * BR + lax.broadcasted_iota(jnp.int32, (BR, 1), 0)
    acc = jnp.where(lane16 == (K - 1), rowv, acc)
    idx_ref[...] = acc


def _knn_idx(emb):
    return pl.pallas_call(
        _topk_body,
        grid=(N // BR,),
        in_specs=[
            pl.BlockSpec((BR, D), lambda i: (i, 0)),
            pl.BlockSpec((N, D), lambda i: (0, 0)),
        ],
        out_specs=pl.BlockSpec((BR, K), lambda i: (i, 0)),
        out_shape=jax.ShapeDtypeStruct((N, K), jnp.int32),
        scratch_shapes=[pltpu.VMEM((1, N), jnp.float32)],
    )(emb, emb)


def _table_body(emb_ref, sw_ref, tab_ref):
    tab_ref[...] = jnp.concatenate(
        [emb_ref[...], sw_ref[...], jnp.zeros((N, 128 - D - 1), jnp.float32)],
        axis=1)


def _build_table(emb, sw2):
    return pl.pallas_call(
        _table_body,
        in_specs=[
            pl.BlockSpec((N, D), lambda: (0, 0)),
            pl.BlockSpec((N, 1), lambda: (0, 0)),
        ],
        out_specs=pl.BlockSpec((N, 128), lambda: (0, 0)),
        out_shape=jax.ShapeDtypeStruct((N, 128), jnp.float32),
    )(emb, sw2)


def _sc_gather(tab, gidx):
    mesh = plsc.VectorSubcoreMesh(
        core_axis_name="core", subcore_axis_name="subcore")

    @functools.partial(
        pl.kernel,
        out_type=jax.ShapeDtypeStruct((N * K, 128), jnp.float32),
        mesh=mesh,
    )
    def k(tab_hbm, gidx_hbm, g_hbm):
        def body(gi_vmem, go_vmem):
            pltpu.sync_copy(tab_hbm.at[gi_vmem.at[0]], go_vmem)

        pltpu.emit_pipeline(
            body,
            grid=(N * K // W,),
            in_specs=[pl.BlockSpec((1, W), lambda i: (0, i))],
            out_specs=[pl.BlockSpec((W, 128), lambda i: (i, 0))],
            core_axis_name=("core", "subcore"),
            dimension_semantics=(pltpu.PARALLEL,),
        )(gidx_hbm, g_hbm)

    return k(tab, gidx)


BA = 256


def _asm_body(emb_ref, g_ref, out_ref):
    eb = emb_ref[...]
    rows = lax.broadcasted_iota(jnp.int32, (BA, BA // K), 0)
    cols = lax.broadcasted_iota(jnp.int32, (BA, BA // K), 1)
    rep = (rows // K == cols).astype(jnp.float32)
    node = lax.dot_general(
        rep, eb, (((1,), (0,)), ((), ())), preferred_element_type=jnp.float32,
        precision=lax.Precision.HIGHEST)
    gb = g_ref[...]
    out_ref[...] = jnp.concatenate(
        [node, gb[:, 0:D], gb[:, D:D + 1]], axis=1)


def _assemble(emb, g):
    return pl.pallas_call(
        _asm_body,
        grid=(N * K // BA,),
        in_specs=[
            pl.BlockSpec((BA // K, D), lambda i: (i, 0)),
            pl.BlockSpec((BA, 128), lambda i: (i, 0)),
        ],
        out_specs=pl.BlockSpec((BA, 2 * D + 1), lambda i: (i, 0)),
        out_shape=jax.ShapeDtypeStruct((N * K, 2 * D + 1), jnp.float32),
    )(emb, g)


def kernel(emb_in, sum_weights):
    idx = _knn_idx(emb_in)
    gidx = idx.reshape(1, N * K)
    tab = _build_table(emb_in, sum_weights.reshape(N, 1))
    g = _sc_gather(tab, gidx)
    return _assemble(emb_in, g)


# speedup vs baseline: 5.9968x; 1.5082x over previous
import functools

import jax
import jax.numpy as jnp
from jax import lax
from jax.experimental import pallas as pl
from jax.experimental.pallas import tpu as pltpu
from jax.experimental.pallas import tpu_sc as plsc

N = 8192
D = 64
K = 16
BR = 256
W = 128


_NET8 = [(0, 1), (2, 3), (4, 5), (6, 7), (0, 2), (1, 3), (4, 6), (5, 7),
         (1, 2), (5, 6), (0, 4), (1, 5), (2, 6), (3, 7), (2, 4), (3, 5),
         (1, 2), (3, 4), (5, 6)]
F = 8
SW = N // F


def _topk_body(q_ref, k_ref, idx_ref, csq_ref):
    i = pl.program_id(0)
    km = k_ref[...]

    @pl.when(i == 0)
    def _():
        csq_ref[...] = jnp.sum(km * km, axis=1)[None, :]

    q = q_ref[...]
    rsq = jnp.sum(q * q, axis=1, keepdims=True)
    dot = lax.dot_general(
        q.astype(jnp.bfloat16), km.astype(jnp.bfloat16),
        (((1,), (1,)), ((), ())), preferred_element_type=jnp.float32)
    s = (rsq + csq_ref[...]) - 2.0 * dot
    rowv = i * BR + lax.broadcasted_iota(jnp.int32, (BR, 1), 0)
    iota_sw = lax.broadcasted_iota(jnp.int32, (BR, SW), 1)
    inf = jnp.float32(jnp.inf)
    nbig = jnp.int32(N)

    vals, cols = [], []
    for k in range(F):
        ck = iota_sw + k * SW
        vk = s[:, k * SW:(k + 1) * SW]
        vk = vk + jnp.where(ck == rowv, jnp.float32(1e9), jnp.float32(0.0))
        vals.append(vk)
        cols.append(ck)
    for (ii, jj) in _NET8:
        av, ac, bv, bc = vals[ii], cols[ii], vals[jj], cols[jj]
        swap = (bv < av) | ((bv == av) & (bc < ac))
        vals[ii] = jnp.where(swap, bv, av)
        cols[ii] = jnp.where(swap, bc, ac)
        vals[jj] = jnp.where(swap, av, bv)
        cols[jj] = jnp.where(swap, ac, bc)

    lane16 = lax.broadcasted_iota(jnp.int32, (BR, K), 1)
    acc = jnp.zeros((BR, K), jnp.int32)
    for t in range(K - 1):
        m = jnp.min(vals[0], axis=1, keepdims=True)
        a = jnp.min(jnp.where(vals[0] == m, cols[0], nbig), axis=1,
                    keepdims=True)
        hit = (vals[0] == m) & (cols[0] == a)
        for k in range(F - 1):
            vals[k] = jnp.where(hit, vals[k + 1], vals[k])
            cols[k] = jnp.where(hit, cols[k + 1], cols[k])
        vals[F - 1] = jnp.where(hit, inf, vals[F - 1])
        cols[F - 1] = jnp.where(hit, nbig, cols[F - 1])
        acc = jnp.where(lane16 == t, a, acc)
    acc = jnp.where(lane16 == (K - 1), rowv, acc)
    idx_ref[...] = acc


def _knn_idx(emb):
    return pl.pallas_call(
        _topk_body,
        grid=(N // BR,),
        in_specs=[
            pl.BlockSpec((BR, D), lambda i: (i, 0)),
            pl.BlockSpec((N, D), lambda i: (0, 0)),
        ],
        out_specs=pl.BlockSpec((BR, K), lambda i: (i, 0)),
        out_shape=jax.ShapeDtypeStruct((N, K), jnp.int32),
        scratch_shapes=[pltpu.VMEM((1, N), jnp.float32)],
    )(emb, emb)


def _table_body(emb_ref, sw_ref, tab_ref):
    tab_ref[...] = jnp.concatenate(
        [emb_ref[...], sw_ref[...], jnp.zeros((N, 128 - D - 1), jnp.float32)],
        axis=1)


def _build_table(emb, sw2):
    return pl.pallas_call(
        _table_body,
        in_specs=[
            pl.BlockSpec((N, D), lambda: (0, 0)),
            pl.BlockSpec((N, 1), lambda: (0, 0)),
        ],
        out_specs=pl.BlockSpec((N, 128), lambda: (0, 0)),
        out_shape=jax.ShapeDtypeStruct((N, 128), jnp.float32),
    )(emb, sw2)


def _sc_gather(tab, gidx):
    mesh = plsc.VectorSubcoreMesh(
        core_axis_name="core", subcore_axis_name="subcore")

    @functools.partial(
        pl.kernel,
        out_type=jax.ShapeDtypeStruct((N * K, 128), jnp.float32),
        mesh=mesh,
    )
    def k(tab_hbm, gidx_hbm, g_hbm):
        def body(gi_vmem, go_vmem):
            pltpu.sync_copy(tab_hbm.at[gi_vmem.at[0]], go_vmem)

        pltpu.emit_pipeline(
            body,
            grid=(N * K // W,),
            in_specs=[pl.BlockSpec((1, W), lambda i: (0, i))],
            out_specs=[pl.BlockSpec((W, 128), lambda i: (i, 0))],
            core_axis_name=("core", "subcore"),
            dimension_semantics=(pltpu.PARALLEL,),
        )(gidx_hbm, g_hbm)

    return k(tab, gidx)


BA = 256


def _asm_body(emb_ref, g_ref, out_ref):
    eb = emb_ref[...]
    rows = lax.broadcasted_iota(jnp.int32, (BA, BA // K), 0)
    cols = lax.broadcasted_iota(jnp.int32, (BA, BA // K), 1)
    rep = (rows // K == cols).astype(jnp.float32)
    node = lax.dot_general(
        rep, eb, (((1,), (0,)), ((), ())), preferred_element_type=jnp.float32,
        precision=lax.Precision.HIGHEST)
    gb = g_ref[...]
    out_ref[...] = jnp.concatenate(
        [node, gb[:, 0:D], gb[:, D:D + 1]], axis=1)


def _assemble(emb, g):
    return pl.pallas_call(
        _asm_body,
        grid=(N * K // BA,),
        in_specs=[
            pl.BlockSpec((BA // K, D), lambda i: (i, 0)),
            pl.BlockSpec((BA, 128), lambda i: (i, 0)),
        ],
        out_specs=pl.BlockSpec((BA, 2 * D + 1), lambda i: (i, 0)),
        out_shape=jax.ShapeDtypeStruct((N * K, 2 * D + 1), jnp.float32),
    )(emb, g)


def kernel(emb_in, sum_weights):
    idx = _knn_idx(emb_in)
    gidx = idx.reshape(1, N * K)
    tab = _build_table(emb_in, sum_weights.reshape(N, 1))
    g = _sc_gather(tab, gidx)
    return _assemble(emb_in, g)


# speedup vs baseline: 6.9162x; 1.1533x over previous
import functools

import jax
import jax.numpy as jnp
from jax import lax
from jax.experimental import pallas as pl
from jax.experimental.pallas import tpu as pltpu
from jax.experimental.pallas import tpu_sc as plsc

N = 8192
D = 64
K = 16
BR = 256
W = 128


_NET8 = [(0, 1), (2, 3), (4, 5), (6, 7), (0, 2), (1, 3), (4, 6), (5, 7),
         (1, 2), (5, 6), (0, 4), (1, 5), (2, 6), (3, 7), (2, 4), (3, 5),
         (1, 2), (3, 4), (5, 6)]
F = 8
SW = N // F


def _topk_body(q_ref, k_ref, idx_ref, csq_ref):
    i = pl.program_id(0)
    km = k_ref[...]

    @pl.when(i == 0)
    def _():
        csq_ref[...] = jnp.sum(km * km, axis=1)[None, :]

    q = q_ref[...]
    rsq = jnp.sum(q * q, axis=1, keepdims=True)
    dot = lax.dot_general(
        q.astype(jnp.bfloat16), km.astype(jnp.bfloat16),
        (((1,), (1,)), ((), ())), preferred_element_type=jnp.float32)
    s = (rsq + csq_ref[...]) - 2.0 * dot
    rowv = i * BR + lax.broadcasted_iota(jnp.int32, (BR, 1), 0)
    iota_sw = lax.broadcasted_iota(jnp.int32, (BR, SW), 1)
    inf = jnp.float32(jnp.inf)
    nbig = jnp.int32(N)

    vals, cols = [], []
    for k in range(F):
        ck = iota_sw + k * SW
        vk = s[:, k * SW:(k + 1) * SW]
        vk = vk + jnp.where(ck == rowv, jnp.float32(1e9), jnp.float32(0.0))
        vals.append(vk)
        cols.append(ck)
    for (ii, jj) in _NET8:
        av, ac, bv, bc = vals[ii], cols[ii], vals[jj], cols[jj]
        swap = (bv < av) | ((bv == av) & (bc < ac))
        vals[ii] = jnp.where(swap, bv, av)
        cols[ii] = jnp.where(swap, bc, ac)
        vals[jj] = jnp.where(swap, av, bv)
        cols[jj] = jnp.where(swap, ac, bc)

    lane16 = lax.broadcasted_iota(jnp.int32, (BR, K), 1)
    acc = jnp.zeros((BR, K), jnp.int32)
    for t in range(K - 1):
        m = jnp.min(vals[0], axis=1, keepdims=True)
        a = jnp.min(jnp.where(vals[0] == m, cols[0], nbig), axis=1,
                    keepdims=True)
        hit = (vals[0] == m) & (cols[0] == a)
        for k in range(F - 1):
            vals[k] = jnp.where(hit, vals[k + 1], vals[k])
            cols[k] = jnp.where(hit, cols[k + 1], cols[k])
        vals[F - 1] = jnp.where(hit, inf, vals[F - 1])
        cols[F - 1] = jnp.where(hit, nbig, cols[F - 1])
        acc = jnp.where(lane16 == t, a, acc)
    acc = jnp.where(lane16 == (K - 1), rowv, acc)
    idx_ref[...] = acc


def _knn_idx(emb):
    return pl.pallas_call(
        _topk_body,
        grid=(N // BR,),
        in_specs=[
            pl.BlockSpec((BR, D), lambda i: (i, 0)),
            pl.BlockSpec((N, D), lambda i: (0, 0)),
        ],
        out_specs=pl.BlockSpec((BR, K), lambda i: (i, 0)),
        out_shape=jax.ShapeDtypeStruct((N, K), jnp.int32),
        scratch_shapes=[pltpu.VMEM((1, N), jnp.float32)],
    )(emb, emb)


def _table_body(emb_ref, sw_ref, tab_ref):
    tab_ref[...] = jnp.concatenate(
        [emb_ref[...], sw_ref[...], jnp.zeros((N, 128 - D - 1), jnp.float32)],
        axis=1)


def _build_table(emb, sw2):
    return pl.pallas_call(
        _table_body,
        in_specs=[
            pl.BlockSpec((N, D), lambda: (0, 0)),
            pl.BlockSpec((N, 1), lambda: (0, 0)),
        ],
        out_specs=pl.BlockSpec((N, 128), lambda: (0, 0)),
        out_shape=jax.ShapeDtypeStruct((N, 128), jnp.float32),
    )(emb, sw2)


def _sc_gather(tab, gidx):
    mesh = plsc.VectorSubcoreMesh(
        core_axis_name="core", subcore_axis_name="subcore")

    @functools.partial(
        pl.kernel,
        out_type=jax.ShapeDtypeStruct((N * K, 128), jnp.float32),
        mesh=mesh,
    )
    def k(tab_hbm, gidx_hbm, g_hbm):
        def body(gi_vmem, go_vmem):
            pltpu.sync_copy(tab_hbm.at[gi_vmem.at[0]], go_vmem)

        pltpu.emit_pipeline(
            body,
            grid=(N * K // W,),
            in_specs=[pl.BlockSpec((1, W), lambda i: (0, i))],
            out_specs=[pl.BlockSpec((W, 128), lambda i: (i, 0))],
            core_axis_name=("core", "subcore"),
            dimension_semantics=(pltpu.PARALLEL,),
        )(gidx_hbm, g_hbm)

    return k(tab, gidx)


BA = 1024


def _asm_body(emb_ref, g_ref, out_ref):
    eb = emb_ref[...]
    rows = lax.broadcasted_iota(jnp.int32, (BA, BA // K), 0)
    cols = lax.broadcasted_iota(jnp.int32, (BA, BA // K), 1)
    rep = (rows // K == cols).astype(jnp.float32)
    node = lax.dot_general(
        rep, eb, (((1,), (0,)), ((), ())), preferred_element_type=jnp.float32,
        precision=lax.Precision.HIGHEST)
    gb = g_ref[...]
    out_ref[...] = jnp.concatenate(
        [node, gb[:, 0:D], gb[:, D:D + 1]], axis=1)


def _assemble(emb, g):
    return pl.pallas_call(
        _asm_body,
        grid=(N * K // BA,),
        in_specs=[
            pl.BlockSpec((BA // K, D), lambda i: (i, 0)),
            pl.BlockSpec((BA, 128), lambda i: (i, 0)),
        ],
        out_specs=pl.BlockSpec((BA, 2 * D + 1), lambda i: (i, 0)),
        out_shape=jax.ShapeDtypeStruct((N * K, 2 * D + 1), jnp.float32),
    )(emb, g)


def kernel(emb_in, sum_weights):
    idx = _knn_idx(emb_in)
    gidx = idx.reshape(1, N * K)
    tab = _build_table(emb_in, sum_weights.reshape(N, 1))
    g = _sc_gather(tab, gidx)
    return _assemble(emb_in, g)


# speedup vs baseline: 7.0147x; 1.0142x over previous
import functools

import jax
import jax.numpy as jnp
from jax import lax
from jax.experimental import pallas as pl
from jax.experimental.pallas import tpu as pltpu
from jax.experimental.pallas import tpu_sc as plsc

N = 8192
D = 64
K = 16
BR = 256
W = 128


_NET8 = [p for r in range(8)
         for p in ([(0, 1), (2, 3), (4, 5), (6, 7)] if r % 2 == 0
                   else [(1, 2), (3, 4), (5, 6)])]
F = 8
SW = N // F


def _topk_body(q_ref, k_ref, idx_ref, csq_ref):
    i = pl.program_id(0)
    km = k_ref[...]

    @pl.when(i == 0)
    def _():
        csq_ref[...] = jnp.sum(km * km, axis=1)[None, :]

    q = q_ref[...]
    rsq = jnp.sum(q * q, axis=1, keepdims=True)
    dot = lax.dot_general(
        q.astype(jnp.bfloat16), km.astype(jnp.bfloat16),
        (((1,), (1,)), ((), ())), preferred_element_type=jnp.float32)
    s = (rsq + csq_ref[...]) - 2.0 * dot
    rowv = i * BR + lax.broadcasted_iota(jnp.int32, (BR, 1), 0)
    iota_sw = lax.broadcasted_iota(jnp.int32, (BR, SW), 1)
    inf = jnp.float32(jnp.inf)
    nbig = jnp.int32(N)

    vals, cols = [], []
    for k in range(F):
        ck = iota_sw + k * SW
        vk = s[:, k * SW:(k + 1) * SW]
        vk = vk + jnp.where(ck == rowv, jnp.float32(1e9), jnp.float32(0.0))
        vals.append(vk)
        cols.append(ck)
    for (ii, jj) in _NET8:
        av, ac, bv, bc = vals[ii], cols[ii], vals[jj], cols[jj]
        swap = bv < av
        vals[ii] = jnp.minimum(av, bv)
        cols[ii] = jnp.where(swap, bc, ac)
        vals[jj] = jnp.maximum(av, bv)
        cols[jj] = jnp.where(swap, ac, bc)

    lane16 = lax.broadcasted_iota(jnp.int32, (BR, K), 1)
    acc = jnp.zeros((BR, K), jnp.int32)
    for t in range(K - 1):
        m = jnp.min(vals[0], axis=1, keepdims=True)
        a = jnp.min(jnp.where(vals[0] == m, cols[0], nbig), axis=1,
                    keepdims=True)
        acc = jnp.where(lane16 == t, a, acc)
        if t == K - 2:
            break
        hit = (vals[0] == m) & (cols[0] == a)
        for k in range(F - 1):
            vals[k] = jnp.where(hit, vals[k + 1], vals[k])
            cols[k] = jnp.where(hit, cols[k + 1], cols[k])
        vals[F - 1] = jnp.where(hit, inf, vals[F - 1])
        cols[F - 1] = jnp.where(hit, nbig, cols[F - 1])
    acc = jnp.where(lane16 == (K - 1), rowv, acc)
    idx_ref[...] = acc


def _knn_idx(emb):
    return pl.pallas_call(
        _topk_body,
        grid=(N // BR,),
        in_specs=[
            pl.BlockSpec((BR, D), lambda i: (i, 0)),
            pl.BlockSpec((N, D), lambda i: (0, 0)),
        ],
        out_specs=pl.BlockSpec((BR, K), lambda i: (i, 0)),
        out_shape=jax.ShapeDtypeStruct((N, K), jnp.int32),
        scratch_shapes=[pltpu.VMEM((1, N), jnp.float32)],
    )(emb, emb)


def _table_body(emb_ref, sw_ref, tab_ref):
    tab_ref[...] = jnp.concatenate(
        [emb_ref[...], sw_ref[...], jnp.zeros((N, 128 - D - 1), jnp.float32)],
        axis=1)


def _build_table(emb, sw2):
    return pl.pallas_call(
        _table_body,
        in_specs=[
            pl.BlockSpec((N, D), lambda: (0, 0)),
            pl.BlockSpec((N, 1), lambda: (0, 0)),
        ],
        out_specs=pl.BlockSpec((N, 128), lambda: (0, 0)),
        out_shape=jax.ShapeDtypeStruct((N, 128), jnp.float32),
    )(emb, sw2)


def _sc_gather(tab, gidx):
    mesh = plsc.VectorSubcoreMesh(
        core_axis_name="core", subcore_axis_name="subcore")

    @functools.partial(
        pl.kernel,
        out_type=jax.ShapeDtypeStruct((N * K, 128), jnp.float32),
        mesh=mesh,
    )
    def k(tab_hbm, gidx_hbm, g_hbm):
        def body(gi_vmem, go_vmem):
            pltpu.sync_copy(tab_hbm.at[gi_vmem.at[0]], go_vmem)

        pltpu.emit_pipeline(
            body,
            grid=(N * K // W,),
            in_specs=[pl.BlockSpec((1, W), lambda i: (0, i))],
            out_specs=[pl.BlockSpec((W, 128), lambda i: (i, 0))],
            core_axis_name=("core", "subcore"),
            dimension_semantics=(pltpu.PARALLEL,),
        )(gidx_hbm, g_hbm)

    return k(tab, gidx)


BA = 1024


def _asm_body(emb_ref, g_ref, out_ref):
    eb = emb_ref[...]
    rows = lax.broadcasted_iota(jnp.int32, (BA, BA // K), 0)
    cols = lax.broadcasted_iota(jnp.int32, (BA, BA // K), 1)
    rep = (rows // K == cols).astype(jnp.float32)
    node = lax.dot_general(
        rep, eb, (((1,), (0,)), ((), ())), preferred_element_type=jnp.float32,
        precision=lax.Precision.HIGHEST)
    gb = g_ref[...]
    out_ref[...] = jnp.concatenate(
        [node, gb[:, 0:D], gb[:, D:D + 1]], axis=1)


def _assemble(emb, g):
    return pl.pallas_call(
        _asm_body,
        grid=(N * K // BA,),
        in_specs=[
            pl.BlockSpec((BA // K, D), lambda i: (i, 0)),
            pl.BlockSpec((BA, 128), lambda i: (i, 0)),
        ],
        out_specs=pl.BlockSpec((BA, 2 * D + 1), lambda i: (i, 0)),
        out_shape=jax.ShapeDtypeStruct((N * K, 2 * D + 1), jnp.float32),
    )(emb, g)


def kernel(emb_in, sum_weights):
    idx = _knn_idx(emb_in)
    gidx = idx.reshape(1, N * K)
    tab = _build_table(emb_in, sum_weights.reshape(N, 1))
    g = _sc_gather(tab, gidx)
    return _assemble(emb_in, g)


# speedup vs baseline: 7.1623x; 1.0210x over previous
import functools

import jax
import jax.numpy as jnp
from jax import lax
from jax.experimental import pallas as pl
from jax.experimental.pallas import tpu as pltpu
from jax.experimental.pallas import tpu_sc as plsc

N = 8192
D = 64
K = 16
BR = 256
W = 128


_NET8 = [p for r in range(8)
         for p in ([(0, 1), (2, 3), (4, 5), (6, 7)] if r % 2 == 0
                   else [(1, 2), (3, 4), (5, 6)])]
F = 8
SW = N // F


HN = N // 2


def _topk_body(h, q_ref, k_ref, idx_ref, csq_ref):
    i = pl.program_id(0)
    km = k_ref[...]

    @pl.when(i == 0)
    def _():
        csq_ref[...] = jnp.sum(km * km, axis=1)[None, :]

    q = q_ref[...]
    rsq = jnp.sum(q * q, axis=1, keepdims=True)
    dot = lax.dot_general(
        q.astype(jnp.bfloat16), km.astype(jnp.bfloat16),
        (((1,), (1,)), ((), ())), preferred_element_type=jnp.float32)
    s = (rsq + csq_ref[...]) - 2.0 * dot
    rowv = (h * HN + i * BR) + lax.broadcasted_iota(jnp.int32, (BR, 1), 0)
    iota_sw = lax.broadcasted_iota(jnp.int32, (BR, SW), 1)
    inf = jnp.float32(jnp.inf)
    nbig = jnp.int32(N)

    vals, cols = [], []
    for k in range(F):
        ck = iota_sw + k * SW
        vk = s[:, k * SW:(k + 1) * SW]
        vk = vk + jnp.where(ck == rowv, jnp.float32(1e9), jnp.float32(0.0))
        vals.append(vk)
        cols.append(ck)
    for (ii, jj) in _NET8:
        av, ac, bv, bc = vals[ii], cols[ii], vals[jj], cols[jj]
        swap = bv < av
        vals[ii] = jnp.minimum(av, bv)
        cols[ii] = jnp.where(swap, bc, ac)
        vals[jj] = jnp.maximum(av, bv)
        cols[jj] = jnp.where(swap, ac, bc)

    lane16 = lax.broadcasted_iota(jnp.int32, (BR, K), 1)
    acc = jnp.zeros((BR, K), jnp.int32)
    for t in range(K - 1):
        m = jnp.min(vals[0], axis=1, keepdims=True)
        a = jnp.min(jnp.where(vals[0] == m, cols[0], nbig), axis=1,
                    keepdims=True)
        acc = jnp.where(lane16 == t, a, acc)
        if t == K - 2:
            break
        hit = (vals[0] == m) & (cols[0] == a)
        for k in range(F - 1):
            vals[k] = jnp.where(hit, vals[k + 1], vals[k])
            cols[k] = jnp.where(hit, cols[k + 1], cols[k])
        vals[F - 1] = jnp.where(hit, inf, vals[F - 1])
        cols[F - 1] = jnp.where(hit, nbig, cols[F - 1])
    acc = jnp.where(lane16 == (K - 1), rowv, acc)
    idx_ref[...] = acc


def _knn_idx(emb, h):
    nb = HN // BR
    return pl.pallas_call(
        functools.partial(_topk_body, h),
        grid=(nb,),
        in_specs=[
            pl.BlockSpec((BR, D), lambda i, _h=h: (_h * (HN // BR) + i, 0)),
            pl.BlockSpec((N, D), lambda i: (0, 0)),
        ],
        out_specs=pl.BlockSpec((BR, K), lambda i: (i, 0)),
        out_shape=jax.ShapeDtypeStruct((HN, K), jnp.int32),
        scratch_shapes=[pltpu.VMEM((1, N), jnp.float32)],
    )(emb, emb)


def _table_body(emb_ref, sw_ref, tab_ref):
    tab_ref[...] = jnp.concatenate(
        [emb_ref[...], sw_ref[...], jnp.zeros((N, 128 - D - 1), jnp.float32)],
        axis=1)


def _build_table(emb, sw2):
    return pl.pallas_call(
        _table_body,
        in_specs=[
            pl.BlockSpec((N, D), lambda: (0, 0)),
            pl.BlockSpec((N, 1), lambda: (0, 0)),
        ],
        out_specs=pl.BlockSpec((N, 128), lambda: (0, 0)),
        out_shape=jax.ShapeDtypeStruct((N, 128), jnp.float32),
    )(emb, sw2)


def _sc_gather(tab, gidx):
    mesh = plsc.VectorSubcoreMesh(
        core_axis_name="core", subcore_axis_name="subcore")

    @functools.partial(
        pl.kernel,
        out_type=jax.ShapeDtypeStruct((HN * K, 128), jnp.float32),
        mesh=mesh,
    )
    def k(tab_hbm, gidx_hbm, g_hbm):
        def body(gi_vmem, go_vmem):
            pltpu.sync_copy(tab_hbm.at[gi_vmem.at[0]], go_vmem)

        pltpu.emit_pipeline(
            body,
            grid=(HN * K // W,),
            in_specs=[pl.BlockSpec((1, W), lambda i: (0, i))],
            out_specs=[pl.BlockSpec((W, 128), lambda i: (i, 0))],
            core_axis_name=("core", "subcore"),
            dimension_semantics=(pltpu.PARALLEL,),
        )(gidx_hbm, g_hbm)

    return k(tab, gidx)


BA = 1024


def _asm_body(emb_ref, g_ref, out_ref):
    eb = emb_ref[...]
    rows = lax.broadcasted_iota(jnp.int32, (BA, BA // K), 0)
    cols = lax.broadcasted_iota(jnp.int32, (BA, BA // K), 1)
    rep = (rows // K == cols).astype(jnp.float32)
    node = lax.dot_general(
        rep, eb, (((1,), (0,)), ((), ())), preferred_element_type=jnp.float32,
        precision=lax.Precision.HIGHEST)
    gb = g_ref[...]
    out_ref[...] = jnp.concatenate(
        [node, gb[:, 0:D], gb[:, D:D + 1]], axis=1)


def _assemble(emb, g, h, prev=None):
    nb = HN * K // BA
    args = [emb, g]
    kwargs = {}
    if prev is not None:
        args.append(prev)
        kwargs["input_output_aliases"] = {2: 0}
        in3 = [pl.BlockSpec(memory_space=pl.ANY)]
    else:
        in3 = []

    def body(emb_ref, g_ref, *rest):
        _asm_body(emb_ref, g_ref, rest[-1])

    return pl.pallas_call(
        body,
        grid=(nb,),
        in_specs=[
            pl.BlockSpec((BA // K, D), lambda i, _h=h: (_h * nb + i, 0)),
            pl.BlockSpec((BA, 128), lambda i: (i, 0)),
        ] + in3,
        out_specs=pl.BlockSpec((BA, 2 * D + 1), lambda i, _h=h: (_h * nb + i, 0)),
        out_shape=jax.ShapeDtypeStruct((N * K, 2 * D + 1), jnp.float32),
        **kwargs,
    )(*args)


def kernel(emb_in, sum_weights):
    tab = _build_table(emb_in, sum_weights.reshape(N, 1))
    idx0 = _knn_idx(emb_in, 0)
    g0 = _sc_gather(tab, idx0.reshape(1, HN * K))
    idx1 = _knn_idx(emb_in, 1)
    g1 = _sc_gather(tab, idx1.reshape(1, HN * K))
    out0 = _assemble(emb_in, g0, 0)
    return _assemble(emb_in, g1, 1, prev=out0)


# speedup vs baseline: 8.4010x; 1.1730x over previous
import functools

import jax
import jax.numpy as jnp
from jax import lax
from jax.experimental import pallas as pl
from jax.experimental.pallas import tpu as pltpu
from jax.experimental.pallas import tpu_sc as plsc

N = 8192
D = 64
K = 16
BR = 256
W = 128


_NET8 = [p for r in range(8)
         for p in ([(0, 1), (2, 3), (4, 5), (6, 7)] if r % 2 == 0
                   else [(1, 2), (3, 4), (5, 6)])]
F = 8
SW = N // F


HN = N // 2


def _topk_body(h, q_ref, k_ref, idx_ref, csq_ref):
    i = pl.program_id(0)
    km = k_ref[...]

    @pl.when(i == 0)
    def _():
        csq_ref[...] = jnp.sum(km * km, axis=1)[None, :]

    q = q_ref[...]
    rsq = jnp.sum(q * q, axis=1, keepdims=True)
    dot = lax.dot_general(
        q.astype(jnp.bfloat16), km.astype(jnp.bfloat16),
        (((1,), (1,)), ((), ())), preferred_element_type=jnp.float32)
    s = (rsq + csq_ref[...]) - 2.0 * dot
    rowv = (h * HN + i * BR) + lax.broadcasted_iota(jnp.int32, (BR, 1), 0)
    iota_sw = lax.broadcasted_iota(jnp.int32, (BR, SW), 1)
    inf = jnp.float32(jnp.inf)
    nbig = jnp.int32(N)

    vals, cols = [], []
    for k in range(F):
        ck = iota_sw + k * SW
        vk = s[:, k * SW:(k + 1) * SW]
        vk = vk + jnp.where(ck == rowv, jnp.float32(1e9), jnp.float32(0.0))
        vals.append(vk)
        cols.append(jnp.full((BR, SW), k, jnp.int32))
    for (ii, jj) in _NET8:
        av, ac, bv, bc = vals[ii], cols[ii], vals[jj], cols[jj]
        swap = bv < av
        vals[ii] = jnp.minimum(av, bv)
        cols[ii] = jnp.where(swap, bc, ac)
        vals[jj] = jnp.maximum(av, bv)
        cols[jj] = jnp.where(swap, ac, bc)
    perm = cols[0]
    for k in range(1, F):
        perm = perm | (cols[k] << (3 * k))

    lane16 = lax.broadcasted_iota(jnp.int32, (BR, K), 1)
    acc = jnp.zeros((BR, K), jnp.int32)
    for t in range(K - 1):
        headcol = ((perm & 7) * SW) | iota_sw
        m = jnp.min(vals[0], axis=1, keepdims=True)
        eq = vals[0] == m
        a = jnp.min(jnp.where(eq, headcol, nbig), axis=1, keepdims=True)
        acc = jnp.where(lane16 == t, a, acc)
        if t == K - 2:
            break
        hit = eq & (headcol == a)
        for k in range(F - 1):
            vals[k] = jnp.where(hit, vals[k + 1], vals[k])
        vals[F - 1] = jnp.where(hit, inf, vals[F - 1])
        perm = jnp.where(hit, lax.shift_right_logical(perm, 3), perm)
    acc = jnp.where(lane16 == (K - 1), rowv, acc)
    idx_ref[...] = acc


def _knn_idx(emb, h):
    nb = HN // BR
    return pl.pallas_call(
        functools.partial(_topk_body, h),
        grid=(nb,),
        in_specs=[
            pl.BlockSpec((BR, D), lambda i, _h=h: (_h * (HN // BR) + i, 0)),
            pl.BlockSpec((N, D), lambda i: (0, 0)),
        ],
        out_specs=pl.BlockSpec((BR, K), lambda i: (i, 0)),
        out_shape=jax.ShapeDtypeStruct((HN, K), jnp.int32),
        scratch_shapes=[pltpu.VMEM((1, N), jnp.float32)],
    )(emb, emb)


def _table_body(emb_ref, sw_ref, tab_ref):
    tab_ref[...] = jnp.concatenate(
        [emb_ref[...], sw_ref[...], jnp.zeros((N, 128 - D - 1), jnp.float32)],
        axis=1)


def _build_table(emb, sw2):
    return pl.pallas_call(
        _table_body,
        in_specs=[
            pl.BlockSpec((N, D), lambda: (0, 0)),
            pl.BlockSpec((N, 1), lambda: (0, 0)),
        ],
        out_specs=pl.BlockSpec((N, 128), lambda: (0, 0)),
        out_shape=jax.ShapeDtypeStruct((N, 128), jnp.float32),
    )(emb, sw2)


def _sc_gather(tab, gidx):
    mesh = plsc.VectorSubcoreMesh(
        core_axis_name="core", subcore_axis_name="subcore")

    @functools.partial(
        pl.kernel,
        out_type=jax.ShapeDtypeStruct((HN * K, 128), jnp.float32),
        mesh=mesh,
    )
    def k(tab_hbm, gidx_hbm, g_hbm):
        def body(gi_vmem, go_vmem):
            pltpu.sync_copy(tab_hbm.at[gi_vmem.at[0]], go_vmem)

        pltpu.emit_pipeline(
            body,
            grid=(HN * K // W,),
            in_specs=[pl.BlockSpec((1, W), lambda i: (0, i))],
            out_specs=[pl.BlockSpec((W, 128), lambda i: (i, 0))],
            core_axis_name=("core", "subcore"),
            dimension_semantics=(pltpu.PARALLEL,),
        )(gidx_hbm, g_hbm)

    return k(tab, gidx)


BA = 1024


def _asm_body(emb_ref, g_ref, out_ref):
    eb = emb_ref[...]
    rows = lax.broadcasted_iota(jnp.int32, (BA, BA // K), 0)
    cols = lax.broadcasted_iota(jnp.int32, (BA, BA // K), 1)
    rep = (rows // K == cols).astype(jnp.float32)
    node = lax.dot_general(
        rep, eb, (((1,), (0,)), ((), ())), preferred_element_type=jnp.float32,
        precision=lax.Precision.HIGHEST)
    gb = g_ref[...]
    out_ref[...] = jnp.concatenate(
        [node, gb[:, 0:D], gb[:, D:D + 1]], axis=1)


def _assemble(emb, g, h, prev=None):
    nb = HN * K // BA
    args = [emb, g]
    kwargs = {}
    if prev is not None:
        args.append(prev)
        kwargs["input_output_aliases"] = {2: 0}
        in3 = [pl.BlockSpec(memory_space=pl.ANY)]
    else:
        in3 = []

    def body(emb_ref, g_ref, *rest):
        _asm_body(emb_ref, g_ref, rest[-1])

    return pl.pallas_call(
        body,
        grid=(nb,),
        in_specs=[
            pl.BlockSpec((BA // K, D), lambda i, _h=h: (_h * nb + i, 0)),
            pl.BlockSpec((BA, 128), lambda i: (i, 0)),
        ] + in3,
        out_specs=pl.BlockSpec((BA, 2 * D + 1), lambda i, _h=h: (_h * nb + i, 0)),
        out_shape=jax.ShapeDtypeStruct((N * K, 2 * D + 1), jnp.float32),
        **kwargs,
    )(*args)


def kernel(emb_in, sum_weights):
    tab = _build_table(emb_in, sum_weights.reshape(N, 1))
    idx0 = _knn_idx(emb_in, 0)
    g0 = _sc_gather(tab, idx0.reshape(1, HN * K))
    idx1 = _knn_idx(emb_in, 1)
    g1 = _sc_gather(tab, idx1.reshape(1, HN * K))
    out0 = _assemble(emb_in, g0, 0)
    return _assemble(emb_in, g1, 1, prev=out0)


# speedup vs baseline: 8.6623x; 1.0311x over previous
import functools

import jax
import jax.numpy as jnp
from jax import lax
from jax.experimental import pallas as pl
from jax.experimental.pallas import tpu as pltpu
from jax.experimental.pallas import tpu_sc as plsc

N = 8192
D = 64
K = 16
BR = 256
W = 128


_NET8 = [p for r in range(8)
         for p in ([(0, 1), (2, 3), (4, 5), (6, 7)] if r % 2 == 0
                   else [(1, 2), (3, 4), (5, 6)])]
F = 8
SW = N // F


HN = N // 2


def _topk_body(h, q_ref, k_ref, idx_ref, csq_ref):
    i = pl.program_id(0)
    km = k_ref[...]

    @pl.when(i == 0)
    def _():
        csq_ref[...] = jnp.sum(km * km, axis=1)[None, :]

    q = q_ref[...]
    rsq = jnp.sum(q * q, axis=1, keepdims=True)
    dot = lax.dot_general(
        q.astype(jnp.bfloat16), km.astype(jnp.bfloat16),
        (((1,), (1,)), ((), ())), preferred_element_type=jnp.float32)
    s = (rsq + csq_ref[...]) - 2.0 * dot
    rowv = (h * HN + i * BR) + lax.broadcasted_iota(jnp.int32, (BR, 1), 0)
    iota_sw = lax.broadcasted_iota(jnp.int32, (BR, SW), 1)
    inf = jnp.float32(jnp.inf)
    nbig = jnp.int32(N)

    vals, cols = [], []
    for k in range(F):
        ck = iota_sw + k * SW
        vk = s[:, k * SW:(k + 1) * SW]
        vk = vk + jnp.where(ck == rowv, jnp.float32(1e9), jnp.float32(0.0))
        vals.append(vk)
        cols.append(jnp.full((BR, SW), k, jnp.int32))
    for (ii, jj) in _NET8:
        av, ac, bv, bc = vals[ii], cols[ii], vals[jj], cols[jj]
        swap = bv < av
        vals[ii] = jnp.minimum(av, bv)
        cols[ii] = jnp.where(swap, bc, ac)
        vals[jj] = jnp.maximum(av, bv)
        cols[jj] = jnp.where(swap, ac, bc)
    perm = cols[0]
    for k in range(1, F):
        perm = perm | (cols[k] << (3 * k))

    lane16 = lax.broadcasted_iota(jnp.int32, (BR, K), 1)
    acc = jnp.zeros((BR, K), jnp.int32)
    for t in range(K - 1):
        headcol = ((perm & 7) * SW) | iota_sw
        m = jnp.min(vals[0], axis=1, keepdims=True)
        eq = vals[0] == m
        a = jnp.min(jnp.where(eq, headcol, nbig), axis=1, keepdims=True)
        acc = jnp.where(lane16 == t, a, acc)
        if t == K - 2:
            break
        hit = eq & (headcol == a)
        for k in range(F - 1):
            vals[k] = jnp.where(hit, vals[k + 1], vals[k])
        vals[F - 1] = jnp.where(hit, inf, vals[F - 1])
        perm = jnp.where(hit, lax.shift_right_logical(perm, 3), perm)
    acc = jnp.where(lane16 == (K - 1), rowv, acc)
    idx_ref[...] = acc


def _knn_idx(emb, h):
    nb = HN // BR
    return pl.pallas_call(
        functools.partial(_topk_body, h),
        grid=(nb,),
        in_specs=[
            pl.BlockSpec((BR, D), lambda i, _h=h: (_h * (HN // BR) + i, 0)),
            pl.BlockSpec((N, D), lambda i: (0, 0)),
        ],
        out_specs=pl.BlockSpec((BR, K), lambda i: (i, 0)),
        out_shape=jax.ShapeDtypeStruct((HN, K), jnp.int32),
        scratch_shapes=[pltpu.VMEM((1, N), jnp.float32)],
    )(emb, emb)


def _table_body(emb_ref, sw_ref, tab_ref):
    tab_ref[...] = jnp.concatenate(
        [emb_ref[...], sw_ref[...], jnp.zeros((N, 128 - D - 1), jnp.float32)],
        axis=1)


def _build_table(emb, sw2):
    return pl.pallas_call(
        _table_body,
        in_specs=[
            pl.BlockSpec((N, D), lambda: (0, 0)),
            pl.BlockSpec((N, 1), lambda: (0, 0)),
        ],
        out_specs=pl.BlockSpec((N, 128), lambda: (0, 0)),
        out_shape=jax.ShapeDtypeStruct((N, 128), jnp.float32),
    )(emb, sw2)


def _sc_gather(tab, gidx):
    mesh = plsc.VectorSubcoreMesh(
        core_axis_name="core", subcore_axis_name="subcore")

    @functools.partial(
        pl.kernel,
        out_type=jax.ShapeDtypeStruct((HN * K, 128), jnp.float32),
        mesh=mesh,
    )
    def k(tab_hbm, gidx_hbm, g_hbm):
        def body(gi_vmem, go_vmem):
            pltpu.sync_copy(tab_hbm.at[gi_vmem.at[0]], go_vmem)

        pltpu.emit_pipeline(
            body,
            grid=(HN * K // W,),
            in_specs=[pl.BlockSpec((1, W), lambda i: (0, i))],
            out_specs=[pl.BlockSpec((W, 128), lambda i: (i, 0))],
            core_axis_name=("core", "subcore"),
            dimension_semantics=(pltpu.PARALLEL,),
        )(gidx_hbm, g_hbm)

    return k(tab, gidx)


BA = 2048


def _asm_body(emb_ref, g_ref, out_ref):
    eb = emb_ref[...]
    rows = lax.broadcasted_iota(jnp.int32, (BA, BA // K), 0)
    cols = lax.broadcasted_iota(jnp.int32, (BA, BA // K), 1)
    rep = (rows // K == cols).astype(jnp.float32)
    node = lax.dot_general(
        rep, eb, (((1,), (0,)), ((), ())), preferred_element_type=jnp.float32,
        precision=lax.Precision.HIGHEST)
    gb = g_ref[...]
    out_ref[...] = jnp.concatenate(
        [node, gb[:, 0:D], gb[:, D:D + 1]], axis=1)


def _assemble(emb, g, h, prev=None):
    nb = HN * K // BA
    args = [emb, g]
    kwargs = {}
    if prev is not None:
        args.append(prev)
        kwargs["input_output_aliases"] = {2: 0}
        in3 = [pl.BlockSpec(memory_space=pl.ANY)]
    else:
        in3 = []

    def body(emb_ref, g_ref, *rest):
        _asm_body(emb_ref, g_ref, rest[-1])

    return pl.pallas_call(
        body,
        grid=(nb,),
        in_specs=[
            pl.BlockSpec((BA // K, D), lambda i, _h=h: (_h * nb + i, 0)),
            pl.BlockSpec((BA, 128), lambda i: (i, 0)),
        ] + in3,
        out_specs=pl.BlockSpec((BA, 2 * D + 1), lambda i, _h=h: (_h * nb + i, 0)),
        out_shape=jax.ShapeDtypeStruct((N * K, 2 * D + 1), jnp.float32),
        **kwargs,
    )(*args)


def kernel(emb_in, sum_weights):
    tab = _build_table(emb_in, sum_weights.reshape(N, 1))
    idx0 = _knn_idx(emb_in, 0)
    g0 = _sc_gather(tab, idx0.reshape(1, HN * K))
    idx1 = _knn_idx(emb_in, 1)
    g1 = _sc_gather(tab, idx1.reshape(1, HN * K))
    out0 = _assemble(emb_in, g0, 0)
    return _assemble(emb_in, g1, 1, prev=out0)
